# baseline (speedup 1.0000x reference)
import functools

import jax
import jax.numpy as jnp
from jax import lax
from jax.experimental import pallas as pl
from jax.experimental.pallas import tpu as pltpu
from jax.experimental.pallas import tpu_sc as plsc

VOCAB = 1000
VOCAB_PAD = 1024
D = 1000
QW = 256
NQ = 4
B, T = 1024, 200
N = B * T

NC, NS, L = 2, 16, 16
NW = NC * NS
B_PER_W = N // NW
CHUNK = 128
NCHUNK = B_PER_W // CHUNK


def _row_lse_body(w_ref, out_ref):
    w = w_ref[...]
    m = jnp.max(w, axis=1)
    s = jnp.sum(jnp.exp(w - m[:, None]), axis=1)
    out_ref[...] = jnp.log(s) + m


def _row_lse(w_pad):
    return pl.pallas_call(
        _row_lse_body,
        out_shape=jax.ShapeDtypeStruct((VOCAB_PAD,), jnp.float32),
    )(w_pad)


_MESH = plsc.VectorSubcoreMesh(core_axis_name="c", subcore_axis_name="s")


@functools.partial(
    pl.kernel,
    mesh=_MESH,
    compiler_params=pltpu.CompilerParams(needs_layout_passes=False),
    out_type=[
        jax.ShapeDtypeStruct((D, N), jnp.float32),
        jax.ShapeDtypeStruct((NW * L,), jnp.float32),
    ],
    scratch_types=[
        pltpu.VMEM((B_PER_W,), jnp.int32),
        pltpu.VMEM((B_PER_W,), jnp.int32),
        pltpu.VMEM((VOCAB_PAD,), jnp.float32),
        pltpu.VMEM((CHUNK, QW), jnp.float32),
        pltpu.VMEM((QW, CHUNK), jnp.float32),
        pltpu.VMEM((L,), jnp.float32),
        pltpu.SemaphoreType.DMA,
    ],
)
def _sc_gather(x_hbm, t_hbm, lse_hbm, w0_hbm, w1_hbm, w2_hbm, w3_hbm,
               out_hbm, psum_hbm,
               idx_v, tgt_v, lse_v, rows_v, tbuf_v, acc_v, gsem):
    wid = lax.axis_index("s") * NC + lax.axis_index("c")
    base = wid * B_PER_W
    pltpu.sync_copy(x_hbm.at[pl.ds(base, B_PER_W)], idx_v)
    pltpu.sync_copy(t_hbm.at[pl.ds(base, B_PER_W)], tgt_v)
    pltpu.sync_copy(lse_hbm, lse_v)
    acc_v[...] = jnp.zeros((L,), jnp.float32)
    wq_hbm = (w0_hbm, w1_hbm, w2_hbm, w3_hbm)
    lanes = lax.iota(jnp.int32, L)

    def step(c, carry):
        ioff = c * CHUNK
        idx_sl = idx_v.at[pl.ds(ioff, CHUNK)]
        for k in range(CHUNK // L):
            xv = idx_v[pl.ds(ioff + k * L, L)]
            acc_v[...] = acc_v[...] + plsc.load_gather(lse_v, [xv])
        for q in range(NQ):
            crows = QW if q < NQ - 1 else D - (NQ - 1) * QW
            pltpu.async_copy(wq_hbm[q].at[idx_sl], rows_v, gsem).wait()

            for k in range(CHUNK // L):
                tg = tgt_v[pl.ds(ioff + k * L, L)]
                tq = tg >> 8
                tl = tg & (QW - 1)
                p = plsc.load_gather(rows_v, [lanes + k * L, tl])
                acc_v[...] = acc_v[...] - jnp.where(tq == q, p, 0.0)

            def tr(cc, carry2):
                for j in range(CHUNK // L):
                    v = plsc.load_gather(
                        rows_v, [lanes + j * L, jnp.full((L,), 0, jnp.int32) + cc]
                    )
                    tbuf_v[cc, pl.ds(j * L, L)] = v
                return carry2

            lax.fori_loop(0, crows, tr, 0)
            if crows == QW:
                pltpu.sync_copy(
                    tbuf_v,
                    out_hbm.at[pl.ds(q * QW, QW), pl.ds(base + ioff, CHUNK)],
                )
            else:
                pltpu.sync_copy(
                    tbuf_v.at[pl.ds(0, crows)],
                    out_hbm.at[pl.ds(q * QW, crows), pl.ds(base + ioff, CHUNK)],
                )
        return carry

    lax.fori_loop(0, NCHUNK, step, 0)
    pltpu.sync_copy(acc_v, psum_hbm.at[pl.ds(wid * L, L)])


def kernel(x, targets, W):
    xf = x.reshape(-1)
    tf = targets.reshape(-1)
    w_pad = jnp.pad(W, ((0, VOCAB_PAD - VOCAB), (0, NQ * QW - D)))
    lse = _row_lse(w_pad[:, :D])
    wq = [w_pad[:, q * QW:(q + 1) * QW] for q in range(NQ)]
    out_t, psums = _sc_gather(xf, tf, lse, *wq)
    loss = jnp.sum(psums) / jnp.float32(N)
    return (jnp.transpose(out_t), loss)


# speedup vs baseline: 2.5211x; 2.5211x over previous
import functools

import jax
import jax.numpy as jnp
from jax import lax
from jax.experimental import pallas as pl
from jax.experimental.pallas import tpu as pltpu
from jax.experimental.pallas import tpu_sc as plsc

VOCAB = 1000
VPAD = 1024
D = 1000
B, T = 1024, 200
N = B * T

NC, NS, L = 2, 16, 16
NW = NC * NS
NBLK = D // 8
SCHUNK = 4096
NSC = N // SCHUNK
B_PER_W = N // NW


def _row_lse_body(w_ref, out_ref):
    w = w_ref[...]
    m = jnp.max(w, axis=1)
    s = jnp.sum(jnp.exp(w - m[:, None]), axis=1)
    out_ref[...] = jnp.log(s) + m


def _row_lse(w_pad):
    return pl.pallas_call(
        _row_lse_body,
        out_shape=jax.ShapeDtypeStruct((VPAD,), jnp.float32),
    )(w_pad)


_MESH = plsc.VectorSubcoreMesh(core_axis_name="c", subcore_axis_name="s")


@functools.partial(
    pl.kernel,
    mesh=_MESH,
    compiler_params=pltpu.CompilerParams(needs_layout_passes=False),
    out_type=[
        jax.ShapeDtypeStruct((D, N), jnp.float32),
        jax.ShapeDtypeStruct((NW * L,), jnp.float32),
    ],
    scratch_types=[
        pltpu.VMEM((B_PER_W,), jnp.int32),
        pltpu.VMEM((VPAD,), jnp.float32),
        pltpu.VMEM((8, VPAD), jnp.float32),
        pltpu.VMEM((2, SCHUNK), jnp.int32),
        pltpu.VMEM((2, SCHUNK), jnp.int32),
        pltpu.VMEM((2, 8, SCHUNK), jnp.float32),
        pltpu.SemaphoreType.DMA,
        pltpu.SemaphoreType.DMA,
        pltpu.SemaphoreType.DMA,
        pltpu.SemaphoreType.DMA,
    ],
)
def _sc_gather(x_hbm, t_hbm, lse_hbm, wt_hbm, out_hbm, psum_hbm,
               idxl_v, lse_v, wt_v, xv_v, tv_v, tbuf_v,
               p0sem, p1sem, w0sem, w1sem):
    wid = lax.axis_index("s") * NC + lax.axis_index("c")
    psem = (p0sem, p1sem)
    wsem = (w0sem, w1sem)
    lanes = lax.iota(jnp.int32, L)

    pltpu.sync_copy(x_hbm.at[pl.ds(wid * B_PER_W, B_PER_W)], idxl_v)
    pltpu.sync_copy(lse_hbm, lse_v)

    def lse_step(k, acc):
        xv = idxl_v[pl.ds(k * L, L)]
        return acc + plsc.load_gather(lse_v, [xv])

    acc = lax.fori_loop(0, B_PER_W // L, lse_step, jnp.zeros((L,), jnp.float32))

    def pf(sc, slot, sem):
        off = sc * SCHUNK
        pltpu.async_copy(x_hbm.at[pl.ds(off, SCHUNK)], xv_v.at[slot], sem)
        pltpu.async_copy(t_hbm.at[pl.ds(off, SCHUNK)], tv_v.at[slot], sem)

    pf(0, 0, psem[0])
    pf(1, 1, psem[1])

    nblk_mine = jnp.where(wid < (NBLK % NW), NBLK // NW + 1, NBLK // NW)

    def block_step(bi, acc):
        blk = bi * NW + wid
        pltpu.sync_copy(wt_hbm.at[pl.ds(blk * 8, 8)], wt_v)
        rows_c = [jnp.full((L,), 0, jnp.int32) + (blk * 8 + r) for r in range(8)]
        rfill = [jnp.full((L,), r, jnp.int32) for r in range(8)]

        def group_step(g, acc2):
            for b in range(2):
                sc = g * 2 + b
                @pl.when(g >= 1)
                def _():
                    pltpu.make_async_copy(
                        tbuf_v.at[b],
                        out_hbm.at[pl.ds(blk * 8, 8), pl.ds(0, SCHUNK)],
                        wsem[b],
                    ).wait()
                pltpu.make_async_copy(
                    x_hbm.at[pl.ds(0, SCHUNK)], xv_v.at[b], psem[b]
                ).wait()
                pltpu.make_async_copy(
                    t_hbm.at[pl.ds(0, SCHUNK)], tv_v.at[b], psem[b]
                ).wait()

                def k_step(k, acc3):
                    xvv = xv_v[b, pl.ds(k * L, L)]
                    tvv = tv_v[b, pl.ds(k * L, L)]
                    for r in range(8):
                        v = plsc.load_gather(wt_v, [rfill[r], xvv])
                        tbuf_v[b, r, pl.ds(k * L, L)] = v
                        acc3 = acc3 - jnp.where(tvv == rows_c[r], v, 0.0)
                    return acc3

                acc2 = lax.fori_loop(0, SCHUNK // L, k_step, acc2)
                pltpu.async_copy(
                    tbuf_v.at[b],
                    out_hbm.at[pl.ds(blk * 8, 8), pl.ds(sc * SCHUNK, SCHUNK)],
                    wsem[b],
                )
                pf(lax.rem(sc + 2, NSC), b, psem[b])
            return acc2

        acc = lax.fori_loop(0, NSC // 2, group_step, acc)
        for b in range(2):
            pltpu.make_async_copy(
                tbuf_v.at[b],
                out_hbm.at[pl.ds(blk * 8, 8), pl.ds(0, SCHUNK)],
                wsem[b],
            ).wait()
        return acc

    acc = lax.fori_loop(0, nblk_mine, block_step, acc)

    for b in range(2):
        pltpu.make_async_copy(
            x_hbm.at[pl.ds(0, SCHUNK)], xv_v.at[b], psem[b]
        ).wait()
        pltpu.make_async_copy(
            t_hbm.at[pl.ds(0, SCHUNK)], tv_v.at[b], psem[b]
        ).wait()
    lse_v[pl.ds(0, L)] = acc
    pltpu.sync_copy(lse_v.at[pl.ds(0, L)], psum_hbm.at[pl.ds(wid * L, L)])


def kernel(x, targets, W):
    xf = x.reshape(-1)
    tf = targets.reshape(-1)
    w_pad = jnp.pad(W, ((0, VPAD - VOCAB), (0, 0)))
    lse = _row_lse(w_pad)
    wt_pad = jnp.pad(W.T, ((0, 0), (0, VPAD - VOCAB)))
    out_t, psums = _sc_gather(xf, tf, lse, wt_pad)
    loss = jnp.sum(psums) / jnp.float32(N)
    return (jnp.transpose(out_t), loss)
